# baseline (speedup 1.0000x reference)
import functools

import jax
import jax.numpy as jnp
from jax import lax
from jax.experimental import pallas as pl
from jax.experimental.pallas import tpu as pltpu
from jax.experimental.pallas import tpu_sc as plsc

DIM = 32
K = 512

NC = 2
NS = 16
L = 16
NW = NC * NS

TN = 1024


def _argmin_body(zt_ref, cb_ref, idx_ref):
    dotp = lax.dot_general(
        cb_ref[...], zt_ref[0], (((0,), (0,)), ((), ())),
        preferred_element_type=jnp.float32)
    m = jnp.min(dotp, axis=0, keepdims=True)
    ks = lax.broadcasted_iota(jnp.int32, dotp.shape, 0)
    idx_ref[...] = jnp.min(jnp.where(dotp == m, ks, K), axis=0)


def _tc_argmin(zt, codebook):
    b, _, t = zt.shape
    return pl.pallas_call(
        _argmin_body,
        grid=(b,),
        in_specs=[
            pl.BlockSpec((1, DIM, t), lambda i: (i, 0, 0)),
            pl.BlockSpec((DIM, K), lambda i: (0, 0)),
        ],
        out_specs=pl.BlockSpec((t,), lambda i: (i,)),
        out_shape=jax.ShapeDtypeStruct((b * t,), jnp.int32),
    )(zt, codebook)


def _make_sc_gather(b, t):
    nb = b // NW
    st = t * DIM
    mesh = plsc.VectorSubcoreMesh(core_axis_name="c", subcore_axis_name="s")

    @functools.partial(
        pl.kernel,
        mesh=mesh,
        out_type=jax.ShapeDtypeStruct((b * DIM * t,), jnp.float32),
        compiler_params=pltpu.CompilerParams(needs_layout_passes=False),
        scratch_types=[
            pltpu.VMEM((K,), jnp.float32),
            pltpu.VMEM((st,), jnp.int32),
            pltpu.VMEM((DIM * t,), jnp.float32),
        ],
    )
    def gather_kernel(cb_hbm, idx_hbm, out_hbm, row_v, idx_v, out_v):
        w = lax.axis_index("s") * NC + lax.axis_index("c")
        pltpu.sync_copy(cb_hbm.at[pl.ds(w * K, K)], row_v)
        tpos = lax.iota(jnp.int32, L) * DIM
        for k in range(nb):
            pltpu.sync_copy(idx_hbm.at[pl.ds(k * st, st)], idx_v)
            @pl.loop(0, t // L)
            def _(j):
                t0 = j * L
                for d in range(DIM):
                    pos = tpos + (t0 * DIM + d)
                    iv = plsc.load_gather(idx_v, [pos])
                    out_v[pl.ds(d * t + t0, L)] = plsc.load_gather(row_v, [iv])
            bb = w * nb + k
            pltpu.sync_copy(out_v, out_hbm.at[pl.ds(bb * DIM * t, DIM * t)])

    return gather_kernel


def kernel(z, codebook):
    b, t, _ = z.shape
    zt = jnp.transpose(z, (0, 2, 1))
    idx = _tc_argmin(zt, codebook)
    qf = _make_sc_gather(b, t)(codebook.reshape(-1), idx)
    return jnp.transpose(qf.reshape(b, DIM, t), (0, 2, 1))


# speedup vs baseline: 1.3085x; 1.3085x over previous
import functools

import jax
import jax.numpy as jnp
from jax import lax
from jax.experimental import pallas as pl
from jax.experimental.pallas import tpu as pltpu
from jax.experimental.pallas import tpu_sc as plsc

DIM = 32
K = 512

NC = 2
NS = 16
L = 16
NW = NC * NS

TN = 1024


def _argmin_body(zt_ref, cb_ref, idx_ref):
    dotp = lax.dot_general(
        cb_ref[...], zt_ref[0], (((0,), (0,)), ((), ())),
        preferred_element_type=jnp.float32)
    m = jnp.min(dotp, axis=0, keepdims=True)
    ks = lax.broadcasted_iota(jnp.int32, dotp.shape, 0)
    idx_ref[...] = jnp.min(jnp.where(dotp == m, ks, K), axis=0)


def _tc_argmin(zt, codebook):
    b, _, t = zt.shape
    return pl.pallas_call(
        _argmin_body,
        grid=(b,),
        in_specs=[
            pl.BlockSpec((1, DIM, t), lambda i: (i, 0, 0)),
            pl.BlockSpec((DIM, K), lambda i: (0, 0)),
        ],
        out_specs=pl.BlockSpec((t,), lambda i: (i,)),
        out_shape=jax.ShapeDtypeStruct((b * t,), jnp.int32),
    )(zt, codebook)


def _make_sc_gather(b, t):
    nb = b // NW
    st = t * DIM
    mesh = plsc.VectorSubcoreMesh(core_axis_name="c", subcore_axis_name="s")

    @functools.partial(
        pl.kernel,
        mesh=mesh,
        out_type=jax.ShapeDtypeStruct((b * DIM * t,), jnp.float32),
        compiler_params=pltpu.CompilerParams(needs_layout_passes=False),
        scratch_types=[
            pltpu.VMEM((K,), jnp.float32),
            pltpu.VMEM((st,), jnp.int32),
            pltpu.VMEM((DIM * t,), jnp.float32),
        ],
    )
    def gather_kernel(cb_hbm, idx_hbm, out_hbm, row_v, idx_v, out_v):
        w = lax.axis_index("s") * NC + lax.axis_index("c")
        pltpu.sync_copy(cb_hbm.at[pl.ds(w * K, K)], row_v)
        tpos = lax.iota(jnp.int32, L) * DIM
        for k in range(nb):
            pltpu.sync_copy(idx_hbm.at[pl.ds(k * st, st)], idx_v)
            @plsc.parallel_loop(0, t // L, unroll=2)
            def _(j):
                t0 = j * L
                for d in range(DIM):
                    pos = tpos + (t0 * DIM + d)
                    iv = plsc.load_gather(idx_v, [pos])
                    out_v[pl.ds(d * t + t0, L)] = plsc.load_gather(row_v, [iv])
            bb = w * nb + k
            pltpu.sync_copy(out_v, out_hbm.at[pl.ds(bb * DIM * t, DIM * t)])

    return gather_kernel


def kernel(z, codebook):
    b, t, _ = z.shape
    zt = jnp.transpose(z, (0, 2, 1))
    idx = _tc_argmin(zt, codebook)
    qf = _make_sc_gather(b, t)(codebook.reshape(-1), idx)
    return jnp.transpose(qf.reshape(b, DIM, t), (0, 2, 1))
